# Initial kernel scaffold; baseline (speedup 1.0000x reference)
import jax
import jax.numpy as jnp
from jax.experimental import pallas as pl


def kernel(K, V):
    raise NotImplementedError("write your pallas kernel here")


# speedup vs baseline: 2.3029x; 2.3029x over previous
import functools

import jax
import jax.numpy as jnp
import numpy as np
from jax.experimental import pallas as pl

_NUM_CLUSTERS = 64


def _indices(seq_len: int, n: int) -> np.ndarray:
    return np.linspace(0.0, float(seq_len - 1), n).astype(np.int32)


def _gather_block(k_ref, v_ref, sel_ref, ko_ref, vo_ref):
    s = sel_ref[...]
    ko_ref[...] = jax.lax.dot(k_ref[...], s,
                              preferred_element_type=jnp.float32)
    vo_ref[...] = jax.lax.dot(v_ref[...], s,
                              preferred_element_type=jnp.float32)


def kernel(K, V):
    rows, seq_len = K.shape
    n = min(_NUM_CLUSTERS, seq_len)
    if seq_len == 0 or n >= seq_len:
        return (K, V)

    idx = _indices(seq_len, n)
    sel = np.zeros((seq_len, n), dtype=np.float32)
    sel[idx, np.arange(n)] = 1.0

    br = 256
    grid = (rows // br,)
    out_shape = jax.ShapeDtypeStruct((rows, n), jnp.float32)
    fn = pl.pallas_call(
        _gather_block,
        grid=grid,
        in_specs=[
            pl.BlockSpec((br, seq_len), lambda i: (i, 0)),
            pl.BlockSpec((br, seq_len), lambda i: (i, 0)),
            pl.BlockSpec((seq_len, n), lambda i: (0, 0)),
        ],
        out_specs=[
            pl.BlockSpec((br, n), lambda i: (i, 0)),
            pl.BlockSpec((br, n), lambda i: (i, 0)),
        ],
        out_shape=[out_shape, out_shape],
    )
    ko, vo = fn(K, V, jnp.asarray(sel))
    return (ko, vo)
